# baseline (speedup 1.0000x reference)
import functools

import jax
import jax.numpy as jnp
from jax import lax
from jax.experimental import pallas as pl
from jax.experimental.pallas import tpu as pltpu
from jax.experimental.pallas import tpu_sc as plsc

LANES = 16


TC_BN = 8192


def _utable_tc(tableT, W, b):
    V = tableT.shape[1]
    H = tableT.shape[0]
    BN = TC_BN
    grid = (V + BN - 1) // BN
    R = grid * (BN // 2)

    def body(t_ref, w_ref, b_ref, u_ref, p_ref):
        blk = t_ref[...]
        wb = jnp.broadcast_to(w_ref[...], (H, BN))
        s = jnp.sum(blk * wb, axis=0) + b_ref[0]
        u_ref[...] = jnp.exp(jnp.tanh(s))
        ta = jnp.swapaxes(blk[:, : BN // 2], 0, 1)
        tb = jnp.swapaxes(blk[:, BN // 2:], 0, 1)
        p_ref[:, pl.ds(0, H)] = ta
        p_ref[:, pl.ds(H, H)] = tb

    return pl.pallas_call(
        body,
        grid=grid,
        in_specs=[
            pl.BlockSpec((H, BN), lambda j: (0, j)),
            pl.BlockSpec((H, 1), lambda j: (0, 0)),
            pl.BlockSpec(memory_space=pltpu.SMEM),
        ],
        out_specs=[
            pl.BlockSpec((BN,), lambda j: (j,)),
            pl.BlockSpec((BN // 2, 2 * H), lambda j: (j, 0)),
        ],
        out_shape=[
            jax.ShapeDtypeStruct((V,), jnp.float32),
            jax.ShapeDtypeStruct((R, 2 * H), jnp.float32),
        ],
    )(tableT, W, b)


def _make_sc_kernel(B, L, R64, H, C):
    info = plsc.get_sparse_core_info()
    NC, NS = info.num_cores, info.num_subcores
    NW = NC * NS
    samples_per_w = B // NW
    n_chunks = samples_per_w // C
    toks = C * L
    toks2 = 2 * toks
    HV = H // LANES
    n_ugrp = (L + LANES - 1) // LANES
    n_grp2 = toks2 // LANES
    KB = TC_BN // 2
    LB = TC_BN.bit_length() - 1
    assert n_chunks % 2 == 0 and toks % LANES == 0

    mesh = plsc.VectorSubcoreMesh(core_axis_name="c", subcore_axis_name="s")

    @functools.partial(
        pl.kernel,
        mesh=mesh,
        out_type=jax.ShapeDtypeStruct((B, H), jnp.float32),
        compiler_params=pltpu.CompilerParams(
            use_tc_tiling_on_sc=False, needs_layout_passes=False
        ),
        scratch_types=[
            pltpu.VMEM((toks2,), jnp.int32),
            pltpu.VMEM((toks2,), jnp.int32),
            pltpu.VMEM((toks, H), jnp.float32),
            pltpu.VMEM((toks, H), jnp.float32),
            pltpu.VMEM((toks,), jnp.float32),
            pltpu.VMEM((toks,), jnp.float32),
            pltpu.VMEM((C, H), jnp.float32),
            pltpu.SemaphoreType.DMA,
            pltpu.SemaphoreType.DMA,
            pltpu.SemaphoreType.DMA,
            pltpu.SemaphoreType.DMA,
        ],
    )
    def k(x_ref, table_ref, utab_ref, out_ref,
          idx_v, idx2_v, emb_a, emb_b, u_a, u_b, out_v,
          sem_ra, sem_rb, sem_ua, sem_ub):
        cid = lax.axis_index("c")
        sid = lax.axis_index("s")
        wid = sid * NC + cid
        sample0 = wid * samples_per_w

        def compute(emb_v, u_v, chunk):
            def sample_body(s, _):
                row0 = s * L
                bases = [min(g * LANES, L - LANES) for g in range(n_ugrp)]
                uvecs = [u_v[pl.ds(row0 + bg, LANES)] for bg in bases]
                acc = [jnp.zeros((LANES,), jnp.float32) for _ in range(HV)]
                dacc = jnp.zeros((LANES,), jnp.float32)
                for l in range(L):
                    g = min(l // LANES, n_ugrp - 1)
                    lane = l - bases[g]
                    u = jnp.broadcast_to(uvecs[g][lane], (LANES,))
                    for j in range(HV):
                        acc[j] = acc[j] + u * emb_v[row0 + l,
                                                    pl.ds(j * LANES, LANES)]
                    dacc = dacc + u
                inv = 1.0 / dacc
                for j in range(HV):
                    out_v[s, pl.ds(j * LANES, LANES)] = acc[j] * inv
                return _

            lax.fori_loop(0, C, sample_body, 0)
            pltpu.sync_copy(out_v, out_ref.at[pl.ds(sample0 + chunk * C, C)])

        def pair_body(i, _):
            c0 = i * 2
            c1 = c0 + 1
            pltpu.sync_copy(
                x_ref.at[pl.ds((sample0 + c0 * C) * L, toks2)], idx_v)
            for g in range(n_grp2):
                sl = pl.ds(g * LANES, LANES)
                v = idx_v[sl]
                hi = lax.shift_left(lax.shift_right_logical(v, LB), LB)
                mid = lax.shift_left(jnp.bitwise_and(v, KB - 1), 1)
                par = jnp.bitwise_and(lax.shift_right_logical(v, LB - 1), 1)
                idx2_v[sl] = jnp.bitwise_or(hi, jnp.bitwise_or(mid, par))
            h_ra = pltpu.async_copy(
                table_ref.at[idx2_v.at[pl.ds(0, toks)]], emb_a, sem_ra)
            h_ua = pltpu.async_copy(
                utab_ref.at[idx_v.at[pl.ds(0, toks)]], u_a, sem_ua)
            h_rb = pltpu.async_copy(
                table_ref.at[idx2_v.at[pl.ds(toks, toks)]], emb_b, sem_rb)
            h_ub = pltpu.async_copy(
                utab_ref.at[idx_v.at[pl.ds(toks, toks)]], u_b, sem_ub)
            h_ra.wait()
            h_ua.wait()
            compute(emb_a, u_a, c0)
            h_rb.wait()
            h_ub.wait()
            compute(emb_b, u_b, c1)
            return _

        lax.fori_loop(0, n_chunks // 2, pair_body, 0)

    return k


def kernel(x, table, W, b):
    B, L = x.shape
    V, H = table.shape
    x_flat = x.reshape(B * L)
    utab, packed = _utable_tc(table.T, W, b)
    packed64 = packed.reshape(packed.shape[0] * 2, H)
    sc = _make_sc_kernel(B, L, packed64.shape[0], H, C=16)
    return sc(x_flat, packed64, utab)


# speedup vs baseline: 1.0480x; 1.0480x over previous
import functools

import jax
import jax.numpy as jnp
from jax import lax
from jax.experimental import pallas as pl
from jax.experimental.pallas import tpu as pltpu
from jax.experimental.pallas import tpu_sc as plsc

LANES = 16


TC_BN = 16384


def _utable_tc(tableT, W, b):
    V = tableT.shape[1]
    H = tableT.shape[0]
    BN = TC_BN
    grid = (V + BN - 1) // BN
    Q = BN // 4
    R = grid * Q
    HW = H // 2

    def body(t_ref, w_ref, b_ref, u_ref, p_ref):
        blk = t_ref[...]
        wb = jnp.broadcast_to(w_ref[...], (H, BN))
        s = jnp.sum(blk * wb, axis=0) + b_ref[0]
        u_ref[...] = jnp.exp(jnp.tanh(s))
        for q in range(4):
            tq = jnp.swapaxes(blk[:, q * Q:(q + 1) * Q], 0, 1)
            tq = tq.astype(jnp.bfloat16)
            lo = lax.bitcast_convert_type(tq[:, :HW],
                                          jnp.uint16).astype(jnp.uint32)
            hi = lax.bitcast_convert_type(tq[:, HW:],
                                          jnp.uint16).astype(jnp.uint32)
            pq = lax.bitcast_convert_type(
                jnp.bitwise_or(lo, lax.shift_left(hi, jnp.uint32(16))), jnp.float32)
            p_ref[:, pl.ds(q * HW, HW)] = pq

    return pl.pallas_call(
        body,
        grid=grid,
        in_specs=[
            pl.BlockSpec((H, BN), lambda j: (0, j)),
            pl.BlockSpec((H, 1), lambda j: (0, 0)),
            pl.BlockSpec(memory_space=pltpu.SMEM),
        ],
        out_specs=[
            pl.BlockSpec((BN,), lambda j: (j,)),
            pl.BlockSpec((Q, 2 * H), lambda j: (j, 0)),
        ],
        out_shape=[
            jax.ShapeDtypeStruct((V,), jnp.float32),
            jax.ShapeDtypeStruct((R, 2 * H), jnp.float32),
        ],
    )(tableT, W, b)


def _make_sc_kernel(B, L, R64, H, C):
    info = plsc.get_sparse_core_info()
    NC, NS = info.num_cores, info.num_subcores
    NW = NC * NS
    samples_per_w = B // NW
    n_chunks = samples_per_w // C
    toks = C * L
    toks2 = 2 * toks
    HV = H // LANES
    n_ugrp = (L + LANES - 1) // LANES
    n_grp2 = toks2 // LANES
    KB4 = TC_BN // 4
    LB = TC_BN.bit_length() - 1
    HW = H // 2
    assert n_chunks % 2 == 0 and toks % LANES == 0

    mesh = plsc.VectorSubcoreMesh(core_axis_name="c", subcore_axis_name="s")

    @functools.partial(
        pl.kernel,
        mesh=mesh,
        out_type=jax.ShapeDtypeStruct((B, H), jnp.float32),
        compiler_params=pltpu.CompilerParams(
            use_tc_tiling_on_sc=False, needs_layout_passes=False
        ),
        scratch_types=[
            pltpu.VMEM((toks2,), jnp.int32),
            pltpu.VMEM((toks2,), jnp.int32),
            pltpu.VMEM((toks, H // 2), jnp.float32),
            pltpu.VMEM((toks, H // 2), jnp.float32),
            pltpu.VMEM((toks,), jnp.float32),
            pltpu.VMEM((toks,), jnp.float32),
            pltpu.VMEM((C, H), jnp.float32),
            pltpu.SemaphoreType.DMA,
            pltpu.SemaphoreType.DMA,
            pltpu.SemaphoreType.DMA,
            pltpu.SemaphoreType.DMA,
        ],
    )
    def k(x_ref, table_ref, utab_ref, out_ref,
          idx_v, idx2_v, emb_a, emb_b, u_a, u_b, out_v,
          sem_ra, sem_rb, sem_ua, sem_ub):
        cid = lax.axis_index("c")
        sid = lax.axis_index("s")
        wid = sid * NC + cid
        sample0 = wid * samples_per_w

        def compute(emb_v, u_v, chunk):
            def sample_body(s, _):
                row0 = s * L
                bases = [min(g * LANES, L - LANES) for g in range(n_ugrp)]
                uvecs = [u_v[pl.ds(row0 + bg, LANES)] for bg in bases]
                acc = [jnp.zeros((LANES,), jnp.float32) for _ in range(HV)]
                dacc = jnp.zeros((LANES,), jnp.float32)
                for l in range(L):
                    g = min(l // LANES, n_ugrp - 1)
                    lane = l - bases[g]
                    u = jnp.broadcast_to(uvecs[g][lane], (LANES,))
                    for j in range(HV // 2):
                        w32 = emb_v[row0 + l, pl.ds(j * LANES, LANES)]
                        e0, e1 = plsc.unpack(
                            plsc.bitcast(w32, jnp.bfloat16),
                            format=plsc.PackFormat.INTERLEAVED)
                        acc[j] = acc[j] + u * e0
                        acc[j + HV // 2] = acc[j + HV // 2] + u * e1
                    dacc = dacc + u
                inv = 1.0 / dacc
                for j in range(HV):
                    out_v[s, pl.ds(j * LANES, LANES)] = acc[j] * inv
                return _

            lax.fori_loop(0, C, sample_body, 0)
            pltpu.sync_copy(out_v, out_ref.at[pl.ds(sample0 + chunk * C, C)])

        def pair_body(i, _):
            c0 = i * 2
            c1 = c0 + 1
            pltpu.sync_copy(
                x_ref.at[pl.ds((sample0 + c0 * C) * L, toks2)], idx_v)
            for g in range(n_grp2):
                sl = pl.ds(g * LANES, LANES)
                v = idx_v[sl]
                hi = lax.shift_left(lax.shift_right_logical(v, LB), LB)
                mid = lax.shift_left(jnp.bitwise_and(v, KB4 - 1), 2)
                par = jnp.bitwise_and(lax.shift_right_logical(v, LB - 2), 3)
                idx2_v[sl] = jnp.bitwise_or(hi, jnp.bitwise_or(mid, par))
            h_ra = pltpu.async_copy(
                table_ref.at[idx2_v.at[pl.ds(0, toks)]], emb_a, sem_ra)
            h_ua = pltpu.async_copy(
                utab_ref.at[idx_v.at[pl.ds(0, toks)]], u_a, sem_ua)
            h_rb = pltpu.async_copy(
                table_ref.at[idx2_v.at[pl.ds(toks, toks)]], emb_b, sem_rb)
            h_ub = pltpu.async_copy(
                utab_ref.at[idx_v.at[pl.ds(toks, toks)]], u_b, sem_ub)
            h_ra.wait()
            h_ua.wait()
            compute(emb_a, u_a, c0)
            h_rb.wait()
            h_ub.wait()
            compute(emb_b, u_b, c1)
            return _

        lax.fori_loop(0, n_chunks // 2, pair_body, 0)

    return k


def kernel(x, table, W, b):
    B, L = x.shape
    V, H = table.shape
    x_flat = x.reshape(B * L)
    utab, packed = _utable_tc(table.T, W, b)
    packed32 = packed.reshape(packed.shape[0] * 4, H // 2)
    sc = _make_sc_kernel(B, L, packed32.shape[0], H, C=16)
    return sc(x_flat, packed32, utab)
